# baseline (speedup 1.0000x reference)
import jax
import jax.numpy as jnp
from jax.experimental import pallas as pl
from jax.experimental.pallas import tpu as pltpu

_N = 16384
_C = 1000
_K = int(_N * 0.7)
_BLOCK = 2048
_GRID = _N // _BLOCK


def _ohem_kernel(logits_ref, labels_ref, loss_ref, acc_ref, losses_scr, corr_scr):
    i = pl.program_id(0)
    x = logits_ref[...]
    lab = labels_ref[...]
    col = jax.lax.broadcasted_iota(jnp.int32, (_BLOCK, _C), 1)
    m = jnp.max(x, axis=1, keepdims=True)
    t = x - m
    e16 = jnp.exp(t.astype(jnp.bfloat16))
    s = jnp.sum(e16, axis=1, keepdims=True).astype(jnp.float32)
    tlab = jnp.sum(jnp.where(col == lab, t, 0.0), axis=1, keepdims=True)
    loss = jnp.log(s) - tlab
    sra = jax.lax.bitcast_convert_type(t, jnp.int32) >> 31
    nm = jnp.float32(_C) + jnp.sum(sra.astype(jnp.float32), axis=1,
                                   keepdims=True)
    corr0 = (tlab == 0.0)
    corr = jnp.sum(jnp.where(corr0, 1.0, 0.0))
    amb = jnp.sum(jnp.where(corr0 & (nm > 1.0), 1.0, 0.0))

    @pl.when(amb > 0.0)
    def _():
        am = jnp.min(jnp.where(t == 0.0, col, _C), axis=1, keepdims=True)
        corr_scr[0, 1] = jnp.sum((am == lab).astype(jnp.float32))

    @pl.when(amb == 0.0)
    def _():
        corr_scr[0, 1] = corr

    losses_scr[pl.ds(i, 1), :] = jnp.transpose(loss, (1, 0))

    @pl.when(i == 0)
    def _():
        corr_scr[0, 0] = corr_scr[0, 1]

    @pl.when(i > 0)
    def _():
        corr_scr[0, 0] = corr_scr[0, 0] + corr_scr[0, 1]

    @pl.when(i == _GRID - 1)
    def _():
        losses = losses_scr[...]
        bits = jax.lax.bitcast_convert_type(losses, jnp.uint32)
        ukey = jnp.where(bits >= jnp.uint32(0x80000000),
                         ~bits, bits | jnp.uint32(0x80000000))

        def body(j, cand):
            cand2 = cand | (jnp.uint32(0x80000000) >> j)
            cnt = jnp.sum((ukey >= cand2).astype(jnp.int32))
            return jnp.where(cnt >= _K, cand2, cand)

        cand = jax.lax.fori_loop(0, 32, body, jnp.uint32(0))
        gt = ukey > cand
        n_gt = jnp.sum(gt.astype(jnp.float32))
        s_gt = jnp.sum(jnp.where(gt, losses, 0.0))
        tbits = jnp.where(cand >= jnp.uint32(0x80000000),
                          cand ^ jnp.uint32(0x80000000), ~cand)
        thr = jax.lax.bitcast_convert_type(tbits, jnp.float32)
        lval = (s_gt + (jnp.float32(_K) - n_gt) * thr) / jnp.float32(_K)
        loss_ref[...] = jnp.full((1, 1), lval, jnp.float32)
        acc_ref[...] = jnp.full((1, 1), corr_scr[0, 0] / jnp.float32(_N),
                                jnp.float32)


def kernel(logits, labels):
    labels2 = labels.reshape(_N, 1).astype(jnp.int32)
    loss, acc = pl.pallas_call(
        _ohem_kernel,
        grid=(_GRID,),
        in_specs=[
            pl.BlockSpec((_BLOCK, _C), lambda i: (i, 0)),
            pl.BlockSpec((_BLOCK, 1), lambda i: (i, 0)),
        ],
        out_specs=[
            pl.BlockSpec((1, 1), lambda i: (0, 0)),
            pl.BlockSpec((1, 1), lambda i: (0, 0)),
        ],
        out_shape=[
            jax.ShapeDtypeStruct((1, 1), jnp.float32),
            jax.ShapeDtypeStruct((1, 1), jnp.float32),
        ],
        scratch_shapes=[
            pltpu.VMEM((_GRID, _BLOCK), jnp.float32),
            pltpu.SMEM((1, 2), jnp.float32),
        ],
        compiler_params=pltpu.CompilerParams(
            dimension_semantics=("arbitrary",),
        ),
    )(logits, labels2)
    return loss[0, 0], acc[0, 0]


# speedup vs baseline: 1.0590x; 1.0590x over previous
import jax
import jax.numpy as jnp
from jax.experimental import pallas as pl
from jax.experimental.pallas import tpu as pltpu

_N = 16384
_C = 1000
_K = int(_N * 0.7)
_BLOCK = 2048
_GRID = _N // _BLOCK


def _ohem_kernel(logits_ref, labels_ref, loss_ref, acc_ref, losses_scr, corr_scr):
    i = pl.program_id(0)
    x = logits_ref[...]
    lab = jnp.transpose(labels_ref[pl.ds(i, 1), :], (1, 0))
    col = jax.lax.broadcasted_iota(jnp.int32, (_BLOCK, _C), 1)
    m = jnp.max(x, axis=1, keepdims=True)
    t = x - m
    s = jnp.sum(jnp.exp(t), axis=1, keepdims=True)
    tlab = jnp.sum(jnp.where(col == lab, t, 0.0), axis=1, keepdims=True)
    loss = jnp.log(s) - tlab
    sra = jax.lax.bitcast_convert_type(t, jnp.int32) >> 31
    nm = jnp.float32(_C) + jnp.sum(sra.astype(jnp.float32), axis=1,
                                   keepdims=True)
    corr0 = (tlab == 0.0)
    corr = jnp.sum(jnp.where(corr0, 1.0, 0.0))
    amb = jnp.sum(jnp.where(corr0 & (nm > 1.0), 1.0, 0.0))

    @pl.when(amb > 0.0)
    def _():
        am = jnp.min(jnp.where(t == 0.0, col, _C), axis=1, keepdims=True)
        corr_scr[0, 1] = jnp.sum((am == lab).astype(jnp.float32))

    @pl.when(amb == 0.0)
    def _():
        corr_scr[0, 1] = corr

    losses_scr[pl.ds(i, 1), :] = jnp.transpose(loss, (1, 0))

    @pl.when(i == 0)
    def _():
        corr_scr[0, 0] = corr_scr[0, 1]

    @pl.when(i > 0)
    def _():
        corr_scr[0, 0] = corr_scr[0, 0] + corr_scr[0, 1]

    @pl.when(i == _GRID - 1)
    def _():
        losses = losses_scr[...]
        bits = jax.lax.bitcast_convert_type(losses, jnp.uint32)
        ukey = jnp.where(bits >= jnp.uint32(0x80000000),
                         ~bits, bits | jnp.uint32(0x80000000))

        def body(j, cand):
            cand2 = cand | (jnp.uint32(0x80000000) >> j)
            cnt = jnp.sum((ukey >= cand2).astype(jnp.int32))
            return jnp.where(cnt >= _K, cand2, cand)

        cand = jax.lax.fori_loop(0, 32, body, jnp.uint32(0))
        gt = ukey > cand
        n_gt = jnp.sum(gt.astype(jnp.float32))
        s_gt = jnp.sum(jnp.where(gt, losses, 0.0))
        tbits = jnp.where(cand >= jnp.uint32(0x80000000),
                          cand ^ jnp.uint32(0x80000000), ~cand)
        thr = jax.lax.bitcast_convert_type(tbits, jnp.float32)
        lval = (s_gt + (jnp.float32(_K) - n_gt) * thr) / jnp.float32(_K)
        loss_ref[...] = jnp.full((1, 1), lval, jnp.float32)
        acc_ref[...] = jnp.full((1, 1), corr_scr[0, 0] / jnp.float32(_N),
                                jnp.float32)


def kernel(logits, labels):
    labels2 = labels.reshape(_GRID, _BLOCK).astype(jnp.int32)
    loss, acc = pl.pallas_call(
        _ohem_kernel,
        grid=(_GRID,),
        in_specs=[
            pl.BlockSpec((_BLOCK, _C), lambda i: (i, 0)),
            pl.BlockSpec((_GRID, _BLOCK), lambda i: (0, 0)),
        ],
        out_specs=[
            pl.BlockSpec((1, 1), lambda i: (0, 0)),
            pl.BlockSpec((1, 1), lambda i: (0, 0)),
        ],
        out_shape=[
            jax.ShapeDtypeStruct((1, 1), jnp.float32),
            jax.ShapeDtypeStruct((1, 1), jnp.float32),
        ],
        scratch_shapes=[
            pltpu.VMEM((_GRID, _BLOCK), jnp.float32),
            pltpu.SMEM((1, 2), jnp.float32),
        ],
        compiler_params=pltpu.CompilerParams(
            dimension_semantics=("arbitrary",),
        ),
    )(logits, labels2)
    return loss[0, 0], acc[0, 0]
